# Initial kernel scaffold; baseline (speedup 1.0000x reference)
import jax
import jax.numpy as jnp
from jax.experimental import pallas as pl


def kernel(input_hidden_states, output_hidden_states, router_hidden_states, Wr, Wd, Wu):
    raise NotImplementedError("write your pallas kernel here")


# speedup vs baseline: 1.0934x; 1.0934x over previous
import functools

import jax
import jax.numpy as jnp
from jax.experimental import pallas as pl
from jax.experimental.pallas import tpu as pltpu


def _gelu_exact(v):
    return 0.5 * v * (1.0 + jax.lax.erf(v * 0.7071067811865476))


def _dense_body(x_ref, out_ref, rh_ref, wr_ref, wd_ref, wu_ref, o_ref, w_scr):
    e = pl.program_id(1)
    ne = pl.num_programs(1)
    T = x_ref.shape[0]
    E = wr_ref.shape[1]

    @pl.when(e == 0)
    def _router():
        logits = jnp.dot(rh_ref[...], wr_ref[...],
                         preferred_element_type=jnp.float32)
        idx = jax.lax.broadcasted_iota(jnp.int32, (T, E), 1)
        m1 = jnp.max(logits, axis=1, keepdims=True)
        i1 = jnp.min(jnp.where(logits == m1, idx, E), axis=1, keepdims=True)
        mask1 = idx == i1
        l2 = jnp.where(mask1, -1e30, logits)
        m2 = jnp.max(l2, axis=1, keepdims=True)
        i2 = jnp.min(jnp.where(l2 == m2, idx, E), axis=1, keepdims=True)
        mask2 = idx == i2
        w1 = 1.0 / (1.0 + jnp.exp(m2 - m1))
        w2 = 1.0 - w1
        w_scr[...] = jnp.where(mask1, w1, 0.0) + jnp.where(mask2, w2, 0.0)

    eidx = jax.lax.broadcasted_iota(jnp.int32, (T, E), 1)
    we = jnp.sum(jnp.where(eidx == e, w_scr[...], 0.0), axis=1,
                 keepdims=True)
    h = jnp.dot(x_ref[...], wd_ref[0], preferred_element_type=jnp.float32)
    h = _gelu_exact(h)
    h = jnp.dot(h, wu_ref[0], preferred_element_type=jnp.float32)
    contrib = h * we

    @pl.when(e == 0)
    def _init():
        o_ref[...] = out_ref[...] + contrib

    @pl.when(e != 0)
    def _acc():
        o_ref[...] = o_ref[...] + contrib


def kernel(input_hidden_states, output_hidden_states, router_hidden_states,
           Wr, Wd, Wu):
    orig_shape = output_hidden_states.shape
    D = orig_shape[-1]
    x = input_hidden_states.reshape(-1, D)
    out = output_hidden_states.reshape(-1, D)
    rh = router_hidden_states.reshape(-1, D)
    N = x.shape[0]
    E = Wr.shape[1]
    A = Wd.shape[2]

    T = 256
    NT = N // T

    grid = (NT, E)
    res = pl.pallas_call(
        _dense_body,
        grid=grid,
        in_specs=[
            pl.BlockSpec((T, D), lambda i, e: (i, 0)),
            pl.BlockSpec((T, D), lambda i, e: (i, 0)),
            pl.BlockSpec((T, D), lambda i, e: (i, 0)),
            pl.BlockSpec((D, E), lambda i, e: (0, 0)),
            pl.BlockSpec((1, D, A), lambda i, e: (e, 0, 0)),
            pl.BlockSpec((1, A, D), lambda i, e: (e, 0, 0)),
        ],
        out_specs=pl.BlockSpec((T, D), lambda i, e: (i, 0)),
        out_shape=jax.ShapeDtypeStruct((N, D), jnp.float32),
        scratch_shapes=[pltpu.VMEM((T, E), jnp.float32)],
    )(x, out, rh, Wr, Wd, Wu)
    return res.reshape(orig_shape)


# speedup vs baseline: 1.6459x; 1.5053x over previous
import functools

import jax
import jax.numpy as jnp
from jax.experimental import pallas as pl
from jax.experimental.pallas import tpu as pltpu


def _gelu_exact(v):
    return 0.5 * v * (1.0 + jax.lax.erf(v * 0.7071067811865476))


def _dense_body(x_ref, out_ref, rh_ref, wr_ref, wd_ref, wu_ref, o_ref, w_scr):
    e = pl.program_id(1)
    ne = pl.num_programs(1)
    T = x_ref.shape[0]
    E = wr_ref.shape[1]

    @pl.when(e == 0)
    def _router():
        logits = jnp.dot(rh_ref[...], wr_ref[...],
                         preferred_element_type=jnp.float32)
        idx = jax.lax.broadcasted_iota(jnp.int32, (T, E), 1)
        m1 = jnp.max(logits, axis=1, keepdims=True)
        i1 = jnp.min(jnp.where(logits == m1, idx, E), axis=1, keepdims=True)
        mask1 = idx == i1
        l2 = jnp.where(mask1, -1e30, logits)
        m2 = jnp.max(l2, axis=1, keepdims=True)
        i2 = jnp.min(jnp.where(l2 == m2, idx, E), axis=1, keepdims=True)
        mask2 = idx == i2
        w1 = 1.0 / (1.0 + jnp.exp(m2 - m1))
        w2 = 1.0 - w1
        w_scr[...] = jnp.where(mask1, w1, 0.0) + jnp.where(mask2, w2, 0.0)

    eidx = jax.lax.broadcasted_iota(jnp.int32, (T, E), 1)
    we = jnp.sum(jnp.where(eidx == e, w_scr[...], 0.0), axis=1,
                 keepdims=True)
    h = jnp.dot(x_ref[...].astype(jnp.bfloat16), wd_ref[0],
                preferred_element_type=jnp.float32)
    h = _gelu_exact(h)
    h = jnp.dot(h.astype(jnp.bfloat16), wu_ref[0],
                preferred_element_type=jnp.float32)
    contrib = h * we

    @pl.when(e == 0)
    def _init():
        o_ref[...] = out_ref[...] + contrib

    @pl.when(e != 0)
    def _acc():
        o_ref[...] = o_ref[...] + contrib


def kernel(input_hidden_states, output_hidden_states, router_hidden_states,
           Wr, Wd, Wu):
    orig_shape = output_hidden_states.shape
    D = orig_shape[-1]
    x = input_hidden_states.reshape(-1, D)
    out = output_hidden_states.reshape(-1, D)
    rh = router_hidden_states.reshape(-1, D)
    N = x.shape[0]
    E = Wr.shape[1]
    A = Wd.shape[2]

    Wd16 = Wd.astype(jnp.bfloat16)
    Wu16 = Wu.astype(jnp.bfloat16)

    T = 512
    NT = N // T

    grid = (NT, E)
    res = pl.pallas_call(
        _dense_body,
        grid=grid,
        in_specs=[
            pl.BlockSpec((T, D), lambda i, e: (i, 0)),
            pl.BlockSpec((T, D), lambda i, e: (i, 0)),
            pl.BlockSpec((T, D), lambda i, e: (i, 0)),
            pl.BlockSpec((D, E), lambda i, e: (0, 0)),
            pl.BlockSpec((1, D, A), lambda i, e: (e, 0, 0)),
            pl.BlockSpec((1, A, D), lambda i, e: (e, 0, 0)),
        ],
        out_specs=pl.BlockSpec((T, D), lambda i, e: (i, 0)),
        out_shape=jax.ShapeDtypeStruct((N, D), jnp.float32),
        scratch_shapes=[pltpu.VMEM((T, E), jnp.float32)],
    )(x, out, rh, Wr, Wd16, Wu16)
    return res.reshape(orig_shape)
